# Initial kernel scaffold; baseline (speedup 1.0000x reference)
import jax
import jax.numpy as jnp
from jax.experimental import pallas as pl


def kernel(edge_attr, emb0, emb1, emb2, W, b):
    raise NotImplementedError("write your pallas kernel here")


# speedup vs baseline: 3.3892x; 3.3892x over previous
import functools

import jax
import jax.numpy as jnp
from jax import lax
from jax.experimental import pallas as pl
from jax.experimental.pallas import tpu as pltpu
from jax.experimental.pallas import tpu_sc as plsc

OUT = 64
NC, NS = 2, 16
NW = NC * NS
SB = 1280
GS = 128
NG = SB // GS
TROWS = 512


def _table_body(emb0_ref, emb1_ref, emb2_ref, w_ref, b_ref, out_ref):
    a = jnp.dot(emb0_ref[...], w_ref[0:64, :], preferred_element_type=jnp.float32)
    bb = jnp.dot(emb1_ref[...], w_ref[64:128, :], preferred_element_type=jnp.float32)
    c = jnp.dot(emb2_ref[...], w_ref[128:192, :], preferred_element_type=jnp.float32)
    r = lax.broadcasted_iota(jnp.int32, (TROWS, 1), 0)
    j32 = lax.broadcasted_iota(jnp.int32, (1, 32), 1)
    j8 = lax.broadcasted_iota(jnp.int32, (1, 8), 1)
    oh0 = ((r // 16) == j32).astype(jnp.float32)
    oh1 = (((r // 2) % 8) == j8).astype(jnp.float32)
    oh2 = ((r % 2) == j8).astype(jnp.float32)
    out_ref[...] = (
        jnp.dot(oh0, a, preferred_element_type=jnp.float32)
        + jnp.dot(oh1, bb, preferred_element_type=jnp.float32)
        + jnp.dot(oh2, c, preferred_element_type=jnp.float32)
        + b_ref[...]
    )


def _build_table(emb0, emb1, emb2, w, b):
    emb0p = jnp.zeros((32, OUT), jnp.float32).at[:emb0.shape[0]].set(emb0)
    emb1p = jnp.zeros((8, OUT), jnp.float32).at[:emb1.shape[0]].set(emb1)
    emb2p = jnp.zeros((8, OUT), jnp.float32).at[:emb2.shape[0]].set(emb2)
    return pl.pallas_call(
        _table_body,
        out_shape=jax.ShapeDtypeStruct((TROWS, OUT), jnp.float32),
    )(emb0p, emb1p, emb2p, w, b.reshape(1, OUT))


def _gather_body(nb, tbl_hbm, c0_hbm, c1_hbm, c2_hbm, out_hbm,
                 c0_v, c1_v, c2_v, idx_v, rows_v, sem):
    wid = lax.axis_index("s") * NC + lax.axis_index("c")
    base = wid * (nb * SB)

    def step(j, carry):
        off = base + j * SB
        pltpu.sync_copy(c0_hbm.at[pl.ds(off, SB)], c0_v)
        pltpu.sync_copy(c1_hbm.at[pl.ds(off, SB)], c1_v)
        pltpu.sync_copy(c2_hbm.at[pl.ds(off, SB)], c2_v)

        def mk(i, carry2):
            s = i * 16
            idx_v[pl.ds(s, 16)] = (
                c0_v[pl.ds(s, 16)] * 16 + c1_v[pl.ds(s, 16)] * 2 + c2_v[pl.ds(s, 16)]
            )
            return carry2

        lax.fori_loop(0, SB // 16, mk, 0, unroll=4)
        for g in range(NG):
            pltpu.async_copy(
                tbl_hbm.at[idx_v.at[pl.ds(g * GS, GS)]],
                rows_v.at[pl.ds(g * GS, GS), :],
                sem,
            )
        for g in range(NG):
            pltpu.make_async_copy(
                tbl_hbm.at[idx_v.at[pl.ds(g * GS, GS)]],
                rows_v.at[pl.ds(g * GS, GS), :],
                sem,
            ).wait()
        pltpu.sync_copy(rows_v, out_hbm.at[pl.ds(off, SB)])
        return carry

    lax.fori_loop(0, nb, step, 0)


def kernel(edge_attr, emb0, emb1, emb2, W, b):
    e = edge_attr.shape[0]
    tile_chunk = NW * SB
    epad = ((e + tile_chunk - 1) // tile_chunk) * tile_chunk
    nb = epad // tile_chunk

    tbl = _build_table(emb0, emb1, emb2, W, b)

    cols = [jnp.pad(edge_attr[:, i], (0, epad - e)) for i in range(3)]

    mesh = plsc.VectorSubcoreMesh(
        core_axis_name="c", subcore_axis_name="s", num_cores=NC, num_subcores=NS
    )
    out = pl.kernel(
        functools.partial(_gather_body, nb),
        out_type=jax.ShapeDtypeStruct((epad, OUT), jnp.float32),
        mesh=mesh,
        compiler_params=pltpu.CompilerParams(use_tc_tiling_on_sc=False),
        scratch_types=[
            pltpu.VMEM((SB,), jnp.int32),
            pltpu.VMEM((SB,), jnp.int32),
            pltpu.VMEM((SB,), jnp.int32),
            pltpu.VMEM((SB,), jnp.int32),
            pltpu.VMEM((SB, OUT), jnp.float32),
            pltpu.SemaphoreType.DMA,
        ],
    )(tbl, cols[0], cols[1], cols[2])
    return out[:e]


# speedup vs baseline: 5.7794x; 1.7052x over previous
import functools

import jax
import jax.numpy as jnp
from jax import lax
from jax.experimental import pallas as pl
from jax.experimental.pallas import tpu as pltpu
from jax.experimental.pallas import tpu_sc as plsc

OUT = 64
NC, NS = 2, 16
NW = NC * NS
SB = 640
GS = 128
NG = SB // GS
TROWS = 512


def _table_body(emb0_ref, emb1_ref, emb2_ref, w_ref, b_ref, out_ref):
    a = jnp.dot(emb0_ref[...], w_ref[0:64, :], preferred_element_type=jnp.float32)
    bb = jnp.dot(emb1_ref[...], w_ref[64:128, :], preferred_element_type=jnp.float32)
    c = jnp.dot(emb2_ref[...], w_ref[128:192, :], preferred_element_type=jnp.float32)
    r = lax.broadcasted_iota(jnp.int32, (TROWS, 1), 0)
    j32 = lax.broadcasted_iota(jnp.int32, (1, 32), 1)
    j8 = lax.broadcasted_iota(jnp.int32, (1, 8), 1)
    oh0 = ((r // 16) == j32).astype(jnp.float32)
    oh1 = (((r // 2) % 8) == j8).astype(jnp.float32)
    oh2 = ((r % 2) == j8).astype(jnp.float32)
    out_ref[...] = (
        jnp.dot(oh0, a, preferred_element_type=jnp.float32)
        + jnp.dot(oh1, bb, preferred_element_type=jnp.float32)
        + jnp.dot(oh2, c, preferred_element_type=jnp.float32)
        + b_ref[...]
    )


def _build_table(emb0, emb1, emb2, w, b):
    emb0p = jnp.zeros((32, OUT), jnp.float32).at[:emb0.shape[0]].set(emb0)
    emb1p = jnp.zeros((8, OUT), jnp.float32).at[:emb1.shape[0]].set(emb1)
    emb2p = jnp.zeros((8, OUT), jnp.float32).at[:emb2.shape[0]].set(emb2)
    return pl.pallas_call(
        _table_body,
        out_shape=jax.ShapeDtypeStruct((TROWS, OUT), jnp.float32),
    )(emb0p, emb1p, emb2p, w, b.reshape(1, OUT))


def _gather_body(nsteps, tbl_hbm, c0_hbm, c1_hbm, c2_hbm, out_hbm,
                 c0_v, c1_v, c2_v, idx_v, rows_v, gsem, wsem):
    wid = lax.axis_index("s") * NC + lax.axis_index("c")
    n_w = (nsteps - wid + NW - 1) // NW

    def step(t, carry):
        j = wid + t * NW
        off = j * SB
        slot = lax.rem(t, 2)
        pltpu.sync_copy(c0_hbm.at[pl.ds(off, SB)], c0_v)
        pltpu.sync_copy(c1_hbm.at[pl.ds(off, SB)], c1_v)
        pltpu.sync_copy(c2_hbm.at[pl.ds(off, SB)], c2_v)

        def mk(i, carry2):
            s = i * 16
            idx_v[pl.ds(s, 16)] = (
                c0_v[pl.ds(s, 16)] * 16 + c1_v[pl.ds(s, 16)] * 2 + c2_v[pl.ds(s, 16)]
            )
            return carry2

        lax.fori_loop(0, SB // 16, mk, 0, unroll=4)

        for g in range(NG):
            pltpu.async_copy(
                tbl_hbm.at[idx_v.at[pl.ds(g * GS, GS)]],
                rows_v.at[slot, pl.ds(g * GS, GS), :],
                gsem,
            )
        for g in range(NG):
            pltpu.make_async_copy(
                tbl_hbm.at[idx_v.at[pl.ds(g * GS, GS)]],
                rows_v.at[slot, pl.ds(g * GS, GS), :],
                gsem,
            ).wait()

        @pl.when(t >= 1)
        def _():
            pltpu.make_async_copy(
                rows_v.at[slot, :, :], out_hbm.at[pl.ds(off, SB)], wsem
            ).wait()

        pltpu.async_copy(rows_v.at[slot, :, :], out_hbm.at[pl.ds(off, SB)], wsem)
        return carry

    lax.fori_loop(0, n_w, step, 0)

    @pl.when(n_w >= 1)
    def _():
        pltpu.make_async_copy(
            rows_v.at[0, :, :], out_hbm.at[pl.ds(0, SB)], wsem
        ).wait()


def kernel(edge_attr, emb0, emb1, emb2, W, b):
    e = edge_attr.shape[0]
    epad = ((e + SB - 1) // SB) * SB
    nsteps = epad // SB

    tbl = _build_table(emb0, emb1, emb2, W, b)

    cols = [edge_attr[:, i] for i in range(3)]
    if epad != e:
        cols = [jnp.pad(c, (0, epad - e)) for c in cols]

    mesh = plsc.VectorSubcoreMesh(
        core_axis_name="c", subcore_axis_name="s", num_cores=NC, num_subcores=NS
    )
    out = pl.kernel(
        functools.partial(_gather_body, nsteps),
        out_type=jax.ShapeDtypeStruct((epad, OUT), jnp.float32),
        mesh=mesh,
        compiler_params=pltpu.CompilerParams(use_tc_tiling_on_sc=False),
        scratch_types=[
            pltpu.VMEM((SB,), jnp.int32),
            pltpu.VMEM((SB,), jnp.int32),
            pltpu.VMEM((SB,), jnp.int32),
            pltpu.VMEM((SB,), jnp.int32),
            pltpu.VMEM((2, SB, OUT), jnp.float32),
            pltpu.SemaphoreType.DMA,
            pltpu.SemaphoreType.DMA,
        ],
    )(tbl, cols[0], cols[1], cols[2])
    return out if epad == e else out[:e]
